# speedup vs baseline: 1.0418x; 1.0418x over previous
import jax
import jax.numpy as jnp
from jax.experimental import pallas as pl
from jax.experimental.pallas import tpu as pltpu

N = 10000
D = 128
H = 64
C = 2000


def _mlp_body(a_ref, b_ref, w1a_ref, w1b_ref, b1_ref, w2_ref, b2_ref, out_ref):
    t = (jnp.dot(a_ref[...], w1a_ref[...], preferred_element_type=jnp.float32)
         + jnp.dot(b_ref[...], w1b_ref[...], preferred_element_type=jnp.float32)
         + b1_ref[...])
    t = jnp.maximum(t, 0.0)
    out_ref[...] = (jnp.dot(t, w2_ref[...], preferred_element_type=jnp.float32)
                    + b2_ref[...])


def _mlp(a, b, w1a, w1b, b1, w2, b2):
    n = a.shape[0]
    return pl.pallas_call(
        _mlp_body,
        out_shape=jax.ShapeDtypeStruct((n, H), jnp.float32),
    )(a, b, w1a, w1b, b1.reshape(1, H), w2, b2.reshape(1, H))


def kernel(x, edge_index, cluster,
           w1_0, b1_0, w2_0, b2_0,
           w1_1, b1_1, w2_1, b2_1,
           w1_2, b1_2, w2_2, b2_2):
    src, dst = edge_index[0], edge_index[1]
    mask = src != dst
    src = jnp.where(mask, src, 0)
    dst = jnp.where(mask, dst, 0)
    loop = jnp.arange(N, dtype=src.dtype)
    src = jnp.concatenate([src, loop])
    dst = jnp.concatenate([dst, loop])

    params = [(w1_0, b1_0, w2_0, b2_0), (w1_1, b1_1, w2_1, b2_1),
              (w1_2, b1_2, w2_2, b2_2)]
    a, b = x[:, :H], x[:, H:]
    for (w1, b1, w2, b2) in params:
        h = _mlp(a, b, w1[:H], w1[H:], b1, w2, b2)
        msgs = h[src]
        aggr = jax.ops.segment_max(msgs, dst, num_segments=N)
        aggr = jnp.where(jnp.isfinite(aggr), aggr, 0.0)
        a, b = h, aggr

    xf = jnp.concatenate([a, b], axis=1)
    pooled = jax.ops.segment_max(xf, cluster, num_segments=C)
    pooled = jnp.where(jnp.isfinite(pooled), pooled, 0.0)
    pooled = pooled / (jnp.linalg.norm(pooled, axis=0) + 1e-12)
    return pooled
